# baseline (speedup 1.0000x reference)
import functools

import jax
import jax.numpy as jnp
from jax import lax
from jax.experimental import pallas as pl
from jax.experimental.pallas import tpu as pltpu
from jax.experimental.pallas import tpu_sc as plsc

NUM_EXPERTS = 8
TOP_K = 2
DIM = 1024
HIDDEN = 2048
TOKENS = 2048
BLK = 128
NBLK = (TOP_K * TOKENS + NUM_EXPERTS * (BLK - 1) + BLK - 1) // BLK
PSLOTS = NBLK * BLK
NW = 32
TPW = TOKENS // NW


def _routing_body(x_ref, gw_ref, dest_ref, sc_ref, be_ref):
    scores = lax.dot_general(
        x_ref[...], gw_ref[...], (((1,), (1,)), ((), ())),
        preferred_element_type=jnp.float32)
    ii = lax.broadcasted_iota(jnp.int32, scores.shape, 1)
    m1 = jnp.max(scores, axis=1, keepdims=True)
    a1 = jnp.min(jnp.where(scores == m1, ii, NUM_EXPERTS), axis=1,
                 keepdims=True)
    oh1 = ii == a1
    masked = jnp.where(oh1, -jnp.inf, scores)
    m2 = jnp.max(masked, axis=1, keepdims=True)
    a2 = jnp.min(jnp.where(masked == m2, ii, NUM_EXPERTS), axis=1,
                 keepdims=True)
    oh2 = ii == a2
    s = jnp.exp(m2 - m1)
    w1 = 1.0 / (1.0 + s)
    sc_ref[...] = jnp.concatenate([w1, 1.0 - w1], axis=1)

    c = (oh1.astype(jnp.float32) + oh2.astype(jnp.float32))
    ri = lax.broadcasted_iota(jnp.int32, (TOKENS, TOKENS), 0)
    ci = lax.broadcasted_iota(jnp.int32, (TOKENS, TOKENS), 1)
    ltri = (ci < ri).astype(jnp.bfloat16)
    s_excl = lax.dot_general(
        ltri, c.astype(jnp.bfloat16), (((1,), (0,)), ((), ())),
        preferred_element_type=jnp.float32)
    counts = jnp.sum(c, axis=0, keepdims=True)
    padded = jnp.ceil(counts / BLK) * BLK
    ei = lax.broadcasted_iota(jnp.int32, (NUM_EXPERTS, NUM_EXPERTS), 0)
    ej = lax.broadcasted_iota(jnp.int32, (NUM_EXPERTS, NUM_EXPERTS), 1)
    l8 = (ei < ej).astype(jnp.float32)
    off = lax.dot_general(padded, l8, (((1,), (0,)), ((), ())),
                          preferred_element_type=jnp.float32)
    offb = jnp.broadcast_to(off, (TOKENS, NUM_EXPERTS))
    rank1 = jnp.sum(jnp.where(oh1, s_excl, 0.0), axis=1, keepdims=True)
    rank2 = jnp.sum(jnp.where(oh2, s_excl, 0.0), axis=1, keepdims=True)
    base1 = jnp.sum(jnp.where(oh1, offb, 0.0), axis=1, keepdims=True)
    base2 = jnp.sum(jnp.where(oh2, offb, 0.0), axis=1, keepdims=True)
    dest1 = (base1 + rank1).astype(jnp.int32)
    dest2 = (base2 + rank2).astype(jnp.int32)
    dest_ref[...] = jnp.concatenate([dest1, dest2], axis=1)

    bi = (lax.broadcasted_iota(jnp.int32, (NBLK, NUM_EXPERTS), 0)
          * BLK).astype(jnp.float32)
    offrow = jnp.broadcast_to(off, (NBLK, NUM_EXPERTS))
    be_ref[...] = (jnp.sum((bi >= offrow).astype(jnp.int32), axis=1,
                           keepdims=True) - 1)


def _tc_routing(x2d, gate_w):
    return pl.pallas_call(
        _routing_body,
        out_shape=[
            jax.ShapeDtypeStruct((TOKENS, 2), jnp.int32),
            jax.ShapeDtypeStruct((TOKENS, 2), jnp.float32),
            jax.ShapeDtypeStruct((NBLK, 1), jnp.int32),
        ],
    )(x2d, gate_w)


def _sc_dispatch(x3, dest1, dest2):
    info = plsc.get_sparse_core_info()
    nc = info.num_cores

    mesh = plsc.VectorSubcoreMesh(core_axis_name="c", subcore_axis_name="s")

    @functools.partial(
        pl.kernel, mesh=mesh,
        out_type=jax.ShapeDtypeStruct((PSLOTS, DIM // 2), jnp.int32),
        scratch_types=[
            pltpu.VMEM((TPW,), jnp.int32),
            pltpu.VMEM((TPW, DIM // 2), jnp.int32),
            pltpu.SemaphoreType.DMA,
        ],
    )
    def k(x3_hbm, d1_hbm, d2_hbm, xs_hbm, idx_v, rows_v, sem):
        wid = lax.axis_index("s") * nc + lax.axis_index("c")
        base = wid * TPW
        pltpu.sync_copy(x3_hbm.at[pl.ds(base, TPW)], rows_v)
        pltpu.sync_copy(d1_hbm.at[pl.ds(base, TPW)], idx_v)
        pltpu.async_copy(rows_v, xs_hbm.at[idx_v], sem).wait()
        pltpu.sync_copy(d2_hbm.at[pl.ds(base, TPW)], idx_v)
        pltpu.async_copy(rows_v, xs_hbm.at[idx_v], sem).wait()

    return k(x3, dest1, dest2)


def _ffn_body(be_ref, xs_ref, w1_ref, w3_ref, w2_ref, out_ref):
    xb = xs_ref[...]
    h1 = lax.dot_general(xb, w1_ref[0], (((1,), (1,)), ((), ())),
                         preferred_element_type=jnp.float32)
    h3 = lax.dot_general(xb, w3_ref[0], (((1,), (1,)), ((), ())),
                         preferred_element_type=jnp.float32)
    g = (h1 * jax.nn.sigmoid(h1) * h3).astype(jnp.bfloat16)
    out_ref[...] = lax.dot_general(g, w2_ref[0], (((1,), (1,)), ((), ())),
                                   preferred_element_type=jnp.float32)


def _tc_ffn(xs2d, w1b, w3b, w2b, be):
    grid_spec = pltpu.PrefetchScalarGridSpec(
        num_scalar_prefetch=1,
        grid=(NBLK,),
        in_specs=[
            pl.BlockSpec((BLK, DIM), lambda b, be_ref: (b, 0)),
            pl.BlockSpec((1, HIDDEN, DIM), lambda b, be_ref: (be_ref[b], 0, 0)),
            pl.BlockSpec((1, HIDDEN, DIM), lambda b, be_ref: (be_ref[b], 0, 0)),
            pl.BlockSpec((1, DIM, HIDDEN), lambda b, be_ref: (be_ref[b], 0, 0)),
        ],
        out_specs=pl.BlockSpec((BLK, DIM), lambda b, be_ref: (b, 0)),
    )
    return pl.pallas_call(
        _ffn_body,
        grid_spec=grid_spec,
        out_shape=jax.ShapeDtypeStruct((PSLOTS, DIM), jnp.float32),
    )(be, xs2d, w1b, w3b, w2b)


def _sc_combine_gather(outs, dest1, dest2):
    info = plsc.get_sparse_core_info()
    nc = info.num_cores

    mesh = plsc.VectorSubcoreMesh(core_axis_name="c", subcore_axis_name="s")

    @functools.partial(
        pl.kernel, mesh=mesh,
        out_type=[
            jax.ShapeDtypeStruct((TOKENS, DIM), jnp.float32),
            jax.ShapeDtypeStruct((TOKENS, DIM), jnp.float32),
        ],
        scratch_types=[
            pltpu.VMEM((TPW,), jnp.int32),
            pltpu.VMEM((TPW, DIM), jnp.float32),
            pltpu.SemaphoreType.DMA,
        ],
    )
    def k(outs_hbm, d1_hbm, d2_hbm, g1_hbm, g2_hbm, idx_v, rows_v, sem):
        wid = lax.axis_index("s") * nc + lax.axis_index("c")
        base = wid * TPW
        pltpu.sync_copy(d1_hbm.at[pl.ds(base, TPW)], idx_v)
        pltpu.async_copy(outs_hbm.at[idx_v], rows_v, sem).wait()
        pltpu.sync_copy(rows_v, g1_hbm.at[pl.ds(base, TPW)])
        pltpu.sync_copy(d2_hbm.at[pl.ds(base, TPW)], idx_v)
        pltpu.async_copy(outs_hbm.at[idx_v], rows_v, sem).wait()
        pltpu.sync_copy(rows_v, g2_hbm.at[pl.ds(base, TPW)])

    return k(outs, dest1, dest2)


def _combine_body(g1_ref, g2_ref, sc_ref, out_ref):
    s = sc_ref[...]
    out_ref[...] = g1_ref[...] * s[:, 0:1] + g2_ref[...] * s[:, 1:2]


def _tc_combine(g1, g2, sc):
    tb = 512
    return pl.pallas_call(
        _combine_body,
        grid=(TOKENS // tb,),
        in_specs=[
            pl.BlockSpec((tb, DIM), lambda t: (t, 0)),
            pl.BlockSpec((tb, DIM), lambda t: (t, 0)),
            pl.BlockSpec((tb, 2), lambda t: (t, 0)),
        ],
        out_specs=pl.BlockSpec((tb, DIM), lambda t: (t, 0)),
        out_shape=jax.ShapeDtypeStruct((TOKENS, DIM), jnp.float32),
    )(g1, g2, sc)


@jax.jit
def kernel(x, gate_w, w1, w2, w3):
    orig_shape = x.shape
    x2d = x.reshape(-1, x.shape[-1])

    dest, sc, be = _tc_routing(x2d, gate_w)
    dest1 = dest[:, 0]
    dest2 = dest[:, 1]

    x_i32 = lax.bitcast_convert_type(
        x2d.astype(jnp.bfloat16).reshape(TOKENS, DIM // 2, 2), jnp.int32)
    xs_i32 = _sc_dispatch(x_i32, dest1, dest2)
    xs2d = lax.bitcast_convert_type(xs_i32, jnp.bfloat16).reshape(PSLOTS, DIM)

    w1b = w1.astype(jnp.bfloat16)
    w3b = w3.astype(jnp.bfloat16)
    w2b = w2.astype(jnp.bfloat16)
    outs = _tc_ffn(xs2d, w1b, w3b, w2b, be.reshape(NBLK))

    g1, g2 = _sc_combine_gather(outs, dest1, dest2)
    y = _tc_combine(g1, g2, sc)
    return y.reshape(orig_shape)


# speedup vs baseline: 1.7223x; 1.7223x over previous
import functools

import jax
import jax.numpy as jnp
from jax import lax
from jax.experimental import pallas as pl
from jax.experimental.pallas import tpu as pltpu
from jax.experimental.pallas import tpu_sc as plsc

NUM_EXPERTS = 8
TOP_K = 2
DIM = 1024
HIDDEN = 2048
TOKENS = 2048
SL = DIM // 128
BLK = 128
NBLK = (TOP_K * TOKENS + NUM_EXPERTS * (BLK - 1) + BLK - 1) // BLK
PSLOTS = NBLK * BLK
NW = 32
TPW = TOKENS // NW


def _routing_body(x_ref, gw_ref, dest_ref, sc_ref, be_ref):
    x = x_ref[...]
    scores = lax.dot_general(
        x, gw_ref[...], (((1,), (1,)), ((), ())),
        preferred_element_type=jnp.float32)
    ii = lax.broadcasted_iota(jnp.int32, scores.shape, 1)
    m1 = jnp.max(scores, axis=1, keepdims=True)
    a1 = jnp.min(jnp.where(scores == m1, ii, NUM_EXPERTS), axis=1,
                 keepdims=True)
    oh1 = ii == a1
    masked = jnp.where(oh1, -jnp.inf, scores)
    m2 = jnp.max(masked, axis=1, keepdims=True)
    a2 = jnp.min(jnp.where(masked == m2, ii, NUM_EXPERTS), axis=1,
                 keepdims=True)
    oh2 = ii == a2
    s = jnp.exp(m2 - m1)
    w1 = 1.0 / (1.0 + s)
    sc_ref[...] = jnp.concatenate([w1, 1.0 - w1], axis=1)

    c = (oh1.astype(jnp.float32) + oh2.astype(jnp.float32))
    ri = lax.broadcasted_iota(jnp.int32, (TOKENS, TOKENS), 0)
    ci = lax.broadcasted_iota(jnp.int32, (TOKENS, TOKENS), 1)
    ltri = (ci < ri).astype(jnp.bfloat16)
    s_excl = lax.dot_general(
        ltri, c.astype(jnp.bfloat16), (((1,), (0,)), ((), ())),
        preferred_element_type=jnp.float32)
    counts = jnp.sum(c, axis=0, keepdims=True)
    padded = jnp.ceil(counts / BLK) * BLK
    ei = lax.broadcasted_iota(jnp.int32, (NUM_EXPERTS, NUM_EXPERTS), 0)
    ej = lax.broadcasted_iota(jnp.int32, (NUM_EXPERTS, NUM_EXPERTS), 1)
    l8 = (ei < ej).astype(jnp.float32)
    off = lax.dot_general(padded, l8, (((1,), (0,)), ((), ())),
                          preferred_element_type=jnp.float32)
    offb = jnp.broadcast_to(off, (TOKENS, NUM_EXPERTS))
    rank1 = jnp.sum(jnp.where(oh1, s_excl, 0.0), axis=1, keepdims=True)
    rank2 = jnp.sum(jnp.where(oh2, s_excl, 0.0), axis=1, keepdims=True)
    base1 = jnp.sum(jnp.where(oh1, offb, 0.0), axis=1, keepdims=True)
    base2 = jnp.sum(jnp.where(oh2, offb, 0.0), axis=1, keepdims=True)
    dest1 = (base1 + rank1).astype(jnp.int32)
    dest2 = (base2 + rank2).astype(jnp.int32)
    dest_ref[...] = jnp.concatenate([dest1, dest2], axis=1)

    bi = (lax.broadcasted_iota(jnp.int32, (NBLK, NUM_EXPERTS), 0)
          * BLK).astype(jnp.float32)
    offrow = jnp.broadcast_to(off, (NBLK, NUM_EXPERTS))
    be_ref[...] = (jnp.sum((bi >= offrow).astype(jnp.int32), axis=1,
                           keepdims=True) - 1)


def _tc_routing(x2d, gate_w):
    return pl.pallas_call(
        _routing_body,
        out_shape=[
            jax.ShapeDtypeStruct((TOKENS, 2), jnp.int32),
            jax.ShapeDtypeStruct((TOKENS, 2), jnp.float32),
            jax.ShapeDtypeStruct((NBLK, 1), jnp.int32),
        ],
    )(x2d, gate_w)


def _sc_dispatch(x2d, dest1, dest2):
    info = plsc.get_sparse_core_info()
    nc = info.num_cores

    mesh = plsc.VectorSubcoreMesh(core_axis_name="c", subcore_axis_name="s")

    @functools.partial(
        pl.kernel, mesh=mesh,
        out_type=jax.ShapeDtypeStruct((PSLOTS, DIM), jnp.float32),
        scratch_types=[
            pltpu.VMEM((TPW,), jnp.int32),
            pltpu.VMEM((TPW, DIM), jnp.float32),
            pltpu.SemaphoreType.DMA,
        ],
    )
    def k(xb_hbm, d1_hbm, d2_hbm, xs_hbm, idx_v, rows_v, sem):
        wid = lax.axis_index("s") * nc + lax.axis_index("c")
        base = wid * TPW
        pltpu.sync_copy(xb_hbm.at[pl.ds(base, TPW)], rows_v)
        pltpu.sync_copy(d1_hbm.at[pl.ds(base, TPW)], idx_v)
        pltpu.async_copy(rows_v, xs_hbm.at[idx_v], sem).wait()
        pltpu.sync_copy(d2_hbm.at[pl.ds(base, TPW)], idx_v)
        pltpu.async_copy(rows_v, xs_hbm.at[idx_v], sem).wait()

    return k(x2d, dest1, dest2)


def _ffn_body(be_ref, xs_ref, w1_ref, w3_ref, w2_ref, out_ref):
    xb = xs_ref[...].astype(jnp.bfloat16)
    w1b = w1_ref[0].astype(jnp.bfloat16)
    w3b = w3_ref[0].astype(jnp.bfloat16)
    w2b = w2_ref[0].astype(jnp.bfloat16)
    h1 = lax.dot_general(xb, w1b, (((1,), (1,)), ((), ())),
                         preferred_element_type=jnp.float32)
    h3 = lax.dot_general(xb, w3b, (((1,), (1,)), ((), ())),
                         preferred_element_type=jnp.float32)
    g = (h1 * jax.nn.sigmoid(h1) * h3).astype(jnp.bfloat16)
    out_ref[...] = lax.dot_general(g, w2b, (((1,), (1,)), ((), ())),
                                   preferred_element_type=jnp.float32)


def _tc_ffn(xs, w1, w3, w2, be):
    grid_spec = pltpu.PrefetchScalarGridSpec(
        num_scalar_prefetch=1,
        grid=(NBLK,),
        in_specs=[
            pl.BlockSpec((BLK, DIM), lambda b, be_ref: (b, 0)),
            pl.BlockSpec((1, HIDDEN, DIM), lambda b, be_ref: (be_ref[b], 0, 0)),
            pl.BlockSpec((1, HIDDEN, DIM), lambda b, be_ref: (be_ref[b], 0, 0)),
            pl.BlockSpec((1, DIM, HIDDEN), lambda b, be_ref: (be_ref[b], 0, 0)),
        ],
        out_specs=pl.BlockSpec((BLK, DIM), lambda b, be_ref: (b, 0)),
    )
    return pl.pallas_call(
        _ffn_body,
        grid_spec=grid_spec,
        out_shape=jax.ShapeDtypeStruct((PSLOTS, DIM), jnp.float32),
    )(be, xs, w1, w3, w2)


def _sc_combine_gather(outs, dest1, dest2):
    info = plsc.get_sparse_core_info()
    nc = info.num_cores

    mesh = plsc.VectorSubcoreMesh(core_axis_name="c", subcore_axis_name="s")

    @functools.partial(
        pl.kernel, mesh=mesh,
        out_type=[
            jax.ShapeDtypeStruct((TOKENS, DIM), jnp.float32),
            jax.ShapeDtypeStruct((TOKENS, DIM), jnp.float32),
        ],
        scratch_types=[
            pltpu.VMEM((TPW,), jnp.int32),
            pltpu.VMEM((TPW, DIM), jnp.float32),
            pltpu.SemaphoreType.DMA,
        ],
    )
    def k(outs_hbm, d1_hbm, d2_hbm, g1_hbm, g2_hbm, idx_v, rows_v, sem):
        wid = lax.axis_index("s") * nc + lax.axis_index("c")
        base = wid * TPW
        pltpu.sync_copy(d1_hbm.at[pl.ds(base, TPW)], idx_v)
        pltpu.async_copy(outs_hbm.at[idx_v], rows_v, sem).wait()
        pltpu.sync_copy(rows_v, g1_hbm.at[pl.ds(base, TPW)])
        pltpu.sync_copy(d2_hbm.at[pl.ds(base, TPW)], idx_v)
        pltpu.async_copy(outs_hbm.at[idx_v], rows_v, sem).wait()
        pltpu.sync_copy(rows_v, g2_hbm.at[pl.ds(base, TPW)])

    return k(outs, dest1, dest2)


def _combine_body(g1_ref, g2_ref, sc_ref, out_ref):
    s = sc_ref[...]
    out_ref[...] = g1_ref[...] * s[:, 0:1] + g2_ref[...] * s[:, 1:2]


def _tc_combine(g1, g2, sc):
    tb = 512
    return pl.pallas_call(
        _combine_body,
        grid=(TOKENS // tb,),
        in_specs=[
            pl.BlockSpec((tb, DIM), lambda t: (t, 0)),
            pl.BlockSpec((tb, DIM), lambda t: (t, 0)),
            pl.BlockSpec((tb, 2), lambda t: (t, 0)),
        ],
        out_specs=pl.BlockSpec((tb, DIM), lambda t: (t, 0)),
        out_shape=jax.ShapeDtypeStruct((TOKENS, DIM), jnp.float32),
    )(g1, g2, sc)


@jax.jit
def kernel(x, gate_w, w1, w2, w3):
    orig_shape = x.shape
    x2d = x.reshape(-1, x.shape[-1])

    dest, sc, be = _tc_routing(x2d, gate_w)
    dest1 = dest[:, 0]
    dest2 = dest[:, 1]

    xs = _sc_dispatch(x2d, dest1, dest2)
    outs = _tc_ffn(xs, w1, w3, w2, be.reshape(NBLK))
    g1, g2 = _sc_combine_gather(outs, dest1, dest2)
    y = _tc_combine(g1, g2, sc)
    return y.reshape(orig_shape)


# speedup vs baseline: 1.8491x; 1.0736x over previous
import functools

import jax
import jax.numpy as jnp
from jax import lax
from jax.experimental import pallas as pl
from jax.experimental.pallas import tpu as pltpu
from jax.experimental.pallas import tpu_sc as plsc

NUM_EXPERTS = 8
TOP_K = 2
DIM = 1024
HIDDEN = 2048
TOKENS = 2048
SL = DIM // 128
BLK = 128
NBLK = (TOP_K * TOKENS + NUM_EXPERTS * (BLK - 1) + BLK - 1) // BLK
PSLOTS = NBLK * BLK
NW = 32
TPW = TOKENS // NW


def _routing_body(x_ref, gw_ref, dest_ref, sc_ref, be_ref, nb_ref):
    x = x_ref[...]
    scores = lax.dot_general(
        x, gw_ref[...], (((1,), (1,)), ((), ())),
        preferred_element_type=jnp.float32)
    ii = lax.broadcasted_iota(jnp.int32, scores.shape, 1)
    m1 = jnp.max(scores, axis=1, keepdims=True)
    a1 = jnp.min(jnp.where(scores == m1, ii, NUM_EXPERTS), axis=1,
                 keepdims=True)
    oh1 = ii == a1
    masked = jnp.where(oh1, -jnp.inf, scores)
    m2 = jnp.max(masked, axis=1, keepdims=True)
    a2 = jnp.min(jnp.where(masked == m2, ii, NUM_EXPERTS), axis=1,
                 keepdims=True)
    oh2 = ii == a2
    s = jnp.exp(m2 - m1)
    w1 = 1.0 / (1.0 + s)
    sc_ref[...] = jnp.concatenate([w1, 1.0 - w1], axis=1)

    c = (oh1.astype(jnp.float32) + oh2.astype(jnp.float32))
    ri = lax.broadcasted_iota(jnp.int32, (TOKENS, TOKENS), 0)
    ci = lax.broadcasted_iota(jnp.int32, (TOKENS, TOKENS), 1)
    ltri = (ci < ri).astype(jnp.bfloat16)
    s_excl = lax.dot_general(
        ltri, c.astype(jnp.bfloat16), (((1,), (0,)), ((), ())),
        preferred_element_type=jnp.float32)
    counts = jnp.sum(c, axis=0, keepdims=True)
    padded = jnp.ceil(counts / BLK) * BLK
    ei = lax.broadcasted_iota(jnp.int32, (NUM_EXPERTS, NUM_EXPERTS), 0)
    ej = lax.broadcasted_iota(jnp.int32, (NUM_EXPERTS, NUM_EXPERTS), 1)
    l8 = (ei < ej).astype(jnp.float32)
    off = lax.dot_general(padded, l8, (((1,), (0,)), ((), ())),
                          preferred_element_type=jnp.float32)
    offb = jnp.broadcast_to(off, (TOKENS, NUM_EXPERTS))
    rank1 = jnp.sum(jnp.where(oh1, s_excl, 0.0), axis=1, keepdims=True)
    rank2 = jnp.sum(jnp.where(oh2, s_excl, 0.0), axis=1, keepdims=True)
    base1 = jnp.sum(jnp.where(oh1, offb, 0.0), axis=1, keepdims=True)
    base2 = jnp.sum(jnp.where(oh2, offb, 0.0), axis=1, keepdims=True)
    dest1 = (base1 + rank1).astype(jnp.int32)
    dest2 = (base2 + rank2).astype(jnp.int32)
    dest_ref[...] = jnp.concatenate([dest1, dest2], axis=1)

    bi = (lax.broadcasted_iota(jnp.int32, (NBLK, NUM_EXPERTS), 0)
          * BLK).astype(jnp.float32)
    offrow = jnp.broadcast_to(off, (NBLK, NUM_EXPERTS))
    be_ref[...] = (jnp.sum((bi >= offrow).astype(jnp.int32), axis=1,
                           keepdims=True) - 1)
    total = jnp.sum(padded)
    nb_ref[...] = (jnp.full((1, 1), 1.0 / BLK) * total).astype(jnp.int32)


def _tc_routing(x2d, gate_w):
    return pl.pallas_call(
        _routing_body,
        out_shape=[
            jax.ShapeDtypeStruct((TOKENS, 2), jnp.int32),
            jax.ShapeDtypeStruct((TOKENS, 2), jnp.float32),
            jax.ShapeDtypeStruct((NBLK, 1), jnp.int32),
            jax.ShapeDtypeStruct((1, 1), jnp.int32),
        ],
    )(x2d, gate_w)


def _sc_dispatch(x2d, dest1, dest2):
    info = plsc.get_sparse_core_info()
    nc = info.num_cores

    mesh = plsc.VectorSubcoreMesh(core_axis_name="c", subcore_axis_name="s")

    @functools.partial(
        pl.kernel, mesh=mesh,
        out_type=jax.ShapeDtypeStruct((PSLOTS, DIM), jnp.float32),
        scratch_types=[
            pltpu.VMEM((TPW,), jnp.int32),
            pltpu.VMEM((TPW, DIM), jnp.float32),
            pltpu.SemaphoreType.DMA,
        ],
    )
    def k(xb_hbm, d1_hbm, d2_hbm, xs_hbm, idx_v, rows_v, sem):
        wid = lax.axis_index("s") * nc + lax.axis_index("c")
        base = wid * TPW
        pltpu.sync_copy(xb_hbm.at[pl.ds(base, TPW)], rows_v)
        pltpu.sync_copy(d1_hbm.at[pl.ds(base, TPW)], idx_v)
        pltpu.async_copy(rows_v, xs_hbm.at[idx_v], sem).wait()
        pltpu.sync_copy(d2_hbm.at[pl.ds(base, TPW)], idx_v)
        pltpu.async_copy(rows_v, xs_hbm.at[idx_v], sem).wait()

    return k(x2d, dest1, dest2)


def _ffn_body(be_ref, nb_ref, xs_ref, w1_ref, w3_ref, w2_ref, out_ref):
    @pl.when(pl.program_id(0) < nb_ref[0])
    def _():
        xb = xs_ref[...].astype(jnp.bfloat16)
        w1b = w1_ref[0].astype(jnp.bfloat16)
        w3b = w3_ref[0].astype(jnp.bfloat16)
        w2b = w2_ref[0].astype(jnp.bfloat16)
        h1 = lax.dot_general(xb, w1b, (((1,), (1,)), ((), ())),
                             preferred_element_type=jnp.float32)
        h3 = lax.dot_general(xb, w3b, (((1,), (1,)), ((), ())),
                             preferred_element_type=jnp.float32)
        g = (h1 * jax.nn.sigmoid(h1) * h3).astype(jnp.bfloat16)
        out_ref[...] = lax.dot_general(g, w2b, (((1,), (1,)), ((), ())),
                                       preferred_element_type=jnp.float32)


def _tc_ffn(xs, w1, w3, w2, be, nb):
    grid_spec = pltpu.PrefetchScalarGridSpec(
        num_scalar_prefetch=2,
        grid=(NBLK,),
        in_specs=[
            pl.BlockSpec((BLK, DIM), lambda b, be_ref, nb_ref: (b, 0)),
            pl.BlockSpec((1, HIDDEN, DIM),
                         lambda b, be_ref, nb_ref: (be_ref[b], 0, 0)),
            pl.BlockSpec((1, HIDDEN, DIM),
                         lambda b, be_ref, nb_ref: (be_ref[b], 0, 0)),
            pl.BlockSpec((1, DIM, HIDDEN),
                         lambda b, be_ref, nb_ref: (be_ref[b], 0, 0)),
        ],
        out_specs=pl.BlockSpec((BLK, DIM), lambda b, be_ref, nb_ref: (b, 0)),
    )
    return pl.pallas_call(
        _ffn_body,
        grid_spec=grid_spec,
        out_shape=jax.ShapeDtypeStruct((PSLOTS, DIM), jnp.float32),
    )(be, nb, xs, w1, w3, w2)


def _sc_combine_gather(outs, dest1, dest2):
    info = plsc.get_sparse_core_info()
    nc = info.num_cores

    mesh = plsc.VectorSubcoreMesh(core_axis_name="c", subcore_axis_name="s")

    @functools.partial(
        pl.kernel, mesh=mesh,
        out_type=[
            jax.ShapeDtypeStruct((TOKENS, DIM), jnp.float32),
            jax.ShapeDtypeStruct((TOKENS, DIM), jnp.float32),
        ],
        scratch_types=[
            pltpu.VMEM((TPW,), jnp.int32),
            pltpu.VMEM((TPW, DIM), jnp.float32),
            pltpu.SemaphoreType.DMA,
        ],
    )
    def k(outs_hbm, d1_hbm, d2_hbm, g1_hbm, g2_hbm, idx_v, rows_v, sem):
        wid = lax.axis_index("s") * nc + lax.axis_index("c")
        base = wid * TPW
        pltpu.sync_copy(d1_hbm.at[pl.ds(base, TPW)], idx_v)
        pltpu.async_copy(outs_hbm.at[idx_v], rows_v, sem).wait()
        pltpu.sync_copy(rows_v, g1_hbm.at[pl.ds(base, TPW)])
        pltpu.sync_copy(d2_hbm.at[pl.ds(base, TPW)], idx_v)
        pltpu.async_copy(outs_hbm.at[idx_v], rows_v, sem).wait()
        pltpu.sync_copy(rows_v, g2_hbm.at[pl.ds(base, TPW)])

    return k(outs, dest1, dest2)


def _combine_body(g1_ref, g2_ref, sc_ref, out_ref):
    s = sc_ref[...]
    out_ref[...] = g1_ref[...] * s[:, 0:1] + g2_ref[...] * s[:, 1:2]


def _tc_combine(g1, g2, sc):
    tb = 512
    return pl.pallas_call(
        _combine_body,
        grid=(TOKENS // tb,),
        in_specs=[
            pl.BlockSpec((tb, DIM), lambda t: (t, 0)),
            pl.BlockSpec((tb, DIM), lambda t: (t, 0)),
            pl.BlockSpec((tb, 2), lambda t: (t, 0)),
        ],
        out_specs=pl.BlockSpec((tb, DIM), lambda t: (t, 0)),
        out_shape=jax.ShapeDtypeStruct((TOKENS, DIM), jnp.float32),
    )(g1, g2, sc)


@jax.jit
def kernel(x, gate_w, w1, w2, w3):
    orig_shape = x.shape
    x2d = x.reshape(-1, x.shape[-1])

    dest, sc, be, nb = _tc_routing(x2d, gate_w)
    dest1 = dest[:, 0]
    dest2 = dest[:, 1]

    xs = _sc_dispatch(x2d, dest1, dest2)
    outs = _tc_ffn(xs, w1, w3, w2, be.reshape(NBLK), nb.reshape(1))
    g1, g2 = _sc_combine_gather(outs, dest1, dest2)
    y = _tc_combine(g1, g2, sc)
    return y.reshape(orig_shape)


# speedup vs baseline: 1.9144x; 1.0353x over previous
import functools

import jax
import jax.numpy as jnp
from jax import lax
from jax.experimental import pallas as pl
from jax.experimental.pallas import tpu as pltpu
from jax.experimental.pallas import tpu_sc as plsc

NUM_EXPERTS = 8
TOP_K = 2
DIM = 1024
HIDDEN = 2048
TOKENS = 2048
SL = DIM // 128
BLK = 128
NBLK = (TOP_K * TOKENS + NUM_EXPERTS * (BLK - 1) + BLK - 1) // BLK
PSLOTS = NBLK * BLK
NW = 32
TPW = TOKENS // NW


def _routing_body(x_ref, gw_ref, dest_ref, wr1_ref, wr2_ref, be_ref, nb_ref):
    x = x_ref[...]
    scores = lax.dot_general(
        x, gw_ref[...], (((1,), (1,)), ((), ())),
        preferred_element_type=jnp.float32)
    ii = lax.broadcasted_iota(jnp.int32, scores.shape, 1)
    m1 = jnp.max(scores, axis=1, keepdims=True)
    a1 = jnp.min(jnp.where(scores == m1, ii, NUM_EXPERTS), axis=1,
                 keepdims=True)
    oh1 = ii == a1
    masked = jnp.where(oh1, -jnp.inf, scores)
    m2 = jnp.max(masked, axis=1, keepdims=True)
    a2 = jnp.min(jnp.where(masked == m2, ii, NUM_EXPERTS), axis=1,
                 keepdims=True)
    oh2 = ii == a2
    s = jnp.exp(m2 - m1)
    w1 = 1.0 / (1.0 + s)
    wr1_ref[...] = jnp.broadcast_to(w1, (TOKENS, 128))
    wr2_ref[...] = jnp.broadcast_to(1.0 - w1, (TOKENS, 128))

    c = (oh1.astype(jnp.float32) + oh2.astype(jnp.float32))
    ri = lax.broadcasted_iota(jnp.int32, (TOKENS, TOKENS), 0)
    ci = lax.broadcasted_iota(jnp.int32, (TOKENS, TOKENS), 1)
    ltri = (ci < ri).astype(jnp.bfloat16)
    s_excl = lax.dot_general(
        ltri, c.astype(jnp.bfloat16), (((1,), (0,)), ((), ())),
        preferred_element_type=jnp.float32)
    counts = jnp.sum(c, axis=0, keepdims=True)
    padded = jnp.ceil(counts / BLK) * BLK
    ei = lax.broadcasted_iota(jnp.int32, (NUM_EXPERTS, NUM_EXPERTS), 0)
    ej = lax.broadcasted_iota(jnp.int32, (NUM_EXPERTS, NUM_EXPERTS), 1)
    l8 = (ei < ej).astype(jnp.float32)
    off = lax.dot_general(padded, l8, (((1,), (0,)), ((), ())),
                          preferred_element_type=jnp.float32)
    offb = jnp.broadcast_to(off, (TOKENS, NUM_EXPERTS))
    rank1 = jnp.sum(jnp.where(oh1, s_excl, 0.0), axis=1, keepdims=True)
    rank2 = jnp.sum(jnp.where(oh2, s_excl, 0.0), axis=1, keepdims=True)
    base1 = jnp.sum(jnp.where(oh1, offb, 0.0), axis=1, keepdims=True)
    base2 = jnp.sum(jnp.where(oh2, offb, 0.0), axis=1, keepdims=True)
    dest1 = (base1 + rank1).astype(jnp.int32)
    dest2 = (base2 + rank2).astype(jnp.int32)
    dest_ref[...] = jnp.concatenate([dest1, dest2], axis=1)

    bi = (lax.broadcasted_iota(jnp.int32, (NBLK, NUM_EXPERTS), 0)
          * BLK).astype(jnp.float32)
    offrow = jnp.broadcast_to(off, (NBLK, NUM_EXPERTS))
    be_ref[...] = (jnp.sum((bi >= offrow).astype(jnp.int32), axis=1,
                           keepdims=True) - 1)
    total = jnp.sum(padded)
    nb_ref[...] = (jnp.full((1, 1), 1.0 / BLK) * total).astype(jnp.int32)


def _tc_routing(x2d, gate_w):
    return pl.pallas_call(
        _routing_body,
        out_shape=[
            jax.ShapeDtypeStruct((TOKENS, 2), jnp.int32),
            jax.ShapeDtypeStruct((TOKENS, 128), jnp.float32),
            jax.ShapeDtypeStruct((TOKENS, 128), jnp.float32),
            jax.ShapeDtypeStruct((NBLK, 1), jnp.int32),
            jax.ShapeDtypeStruct((1, 1), jnp.int32),
        ],
    )(x2d, gate_w)


def _sc_dispatch(x2d, dest1, dest2, wr1, wr2):
    info = plsc.get_sparse_core_info()
    nc = info.num_cores

    mesh = plsc.VectorSubcoreMesh(core_axis_name="c", subcore_axis_name="s")

    @functools.partial(
        pl.kernel, mesh=mesh,
        out_type=[
            jax.ShapeDtypeStruct((PSLOTS, DIM), jnp.float32),
            jax.ShapeDtypeStruct((PSLOTS, 128), jnp.float32),
        ],
        scratch_types=[
            pltpu.VMEM((TPW,), jnp.int32),
            pltpu.VMEM((TPW, DIM), jnp.float32),
            pltpu.VMEM((TPW, 128), jnp.float32),
            pltpu.SemaphoreType.DMA,
            pltpu.SemaphoreType.DMA,
        ],
    )
    def k(xb_hbm, d1_hbm, d2_hbm, w1_hbm, w2_hbm, xs_hbm, ws_hbm,
          idx_v, rows_v, wrow_v, sem, sem2):
        wid = lax.axis_index("s") * nc + lax.axis_index("c")
        base = wid * TPW
        pltpu.sync_copy(xb_hbm.at[pl.ds(base, TPW)], rows_v)
        pltpu.sync_copy(d1_hbm.at[pl.ds(base, TPW)], idx_v)
        pltpu.sync_copy(w1_hbm.at[pl.ds(base, TPW)], wrow_v)
        c1 = pltpu.async_copy(rows_v, xs_hbm.at[idx_v], sem)
        c2 = pltpu.async_copy(wrow_v, ws_hbm.at[idx_v], sem2)
        c1.wait()
        c2.wait()
        pltpu.sync_copy(d2_hbm.at[pl.ds(base, TPW)], idx_v)
        pltpu.sync_copy(w2_hbm.at[pl.ds(base, TPW)], wrow_v)
        c1 = pltpu.async_copy(rows_v, xs_hbm.at[idx_v], sem)
        c2 = pltpu.async_copy(wrow_v, ws_hbm.at[idx_v], sem2)
        c1.wait()
        c2.wait()

    return k(x2d, dest1, dest2, wr1, wr2)


def _ffn_body(be_ref, nb_ref, xs_ref, ws_ref, w1_ref, w3_ref, w2_ref,
              out_ref):
    @pl.when(pl.program_id(0) < nb_ref[0])
    def _():
        xb = xs_ref[...].astype(jnp.bfloat16)
        w1b = w1_ref[0].astype(jnp.bfloat16)
        w3b = w3_ref[0].astype(jnp.bfloat16)
        w2b = w2_ref[0].astype(jnp.bfloat16)
        h1 = lax.dot_general(xb, w1b, (((1,), (1,)), ((), ())),
                             preferred_element_type=jnp.float32)
        h3 = lax.dot_general(xb, w3b, (((1,), (1,)), ((), ())),
                             preferred_element_type=jnp.float32)
        g = (h1 * jax.nn.sigmoid(h1) * h3).astype(jnp.bfloat16)
        out = lax.dot_general(g, w2b, (((1,), (1,)), ((), ())),
                              preferred_element_type=jnp.float32)
        out_ref[...] = out * ws_ref[:, 0:1]


def _tc_ffn(xs, ws, w1, w3, w2, be, nb):
    grid_spec = pltpu.PrefetchScalarGridSpec(
        num_scalar_prefetch=2,
        grid=(NBLK,),
        in_specs=[
            pl.BlockSpec((BLK, DIM), lambda b, be_ref, nb_ref: (b, 0)),
            pl.BlockSpec((BLK, 128), lambda b, be_ref, nb_ref: (b, 0)),
            pl.BlockSpec((1, HIDDEN, DIM),
                         lambda b, be_ref, nb_ref: (be_ref[b], 0, 0)),
            pl.BlockSpec((1, HIDDEN, DIM),
                         lambda b, be_ref, nb_ref: (be_ref[b], 0, 0)),
            pl.BlockSpec((1, DIM, HIDDEN),
                         lambda b, be_ref, nb_ref: (be_ref[b], 0, 0)),
        ],
        out_specs=pl.BlockSpec((BLK, DIM), lambda b, be_ref, nb_ref: (b, 0)),
    )
    return pl.pallas_call(
        _ffn_body,
        grid_spec=grid_spec,
        out_shape=jax.ShapeDtypeStruct((PSLOTS, DIM), jnp.float32),
    )(be, nb, xs, ws, w1, w3, w2)


def _sc_combine_gather(outs, dest1, dest2):
    info = plsc.get_sparse_core_info()
    nc = info.num_cores

    mesh = plsc.VectorSubcoreMesh(core_axis_name="c", subcore_axis_name="s")

    @functools.partial(
        pl.kernel, mesh=mesh,
        out_type=jax.ShapeDtypeStruct((TOKENS, DIM), jnp.float32),
        scratch_types=[
            pltpu.VMEM((TPW,), jnp.int32),
            pltpu.VMEM((TPW, DIM), jnp.float32),
            pltpu.SemaphoreType.DMA,
        ],
    )
    def k(outs_hbm, d1_hbm, d2_hbm, y_hbm, idx_v, rows_v, sem):
        wid = lax.axis_index("s") * nc + lax.axis_index("c")
        base = wid * TPW
        pltpu.sync_copy(d1_hbm.at[pl.ds(base, TPW)], idx_v)
        pltpu.async_copy(outs_hbm.at[idx_v], rows_v, sem).wait()
        pltpu.sync_copy(d2_hbm.at[pl.ds(base, TPW)], idx_v)
        pltpu.async_copy(outs_hbm.at[idx_v], rows_v, sem, add=True).wait()
        pltpu.sync_copy(rows_v, y_hbm.at[pl.ds(base, TPW)])

    return k(outs, dest1, dest2)


@jax.jit
def kernel(x, gate_w, w1, w2, w3):
    orig_shape = x.shape
    x2d = x.reshape(-1, x.shape[-1])

    dest, wr1, wr2, be, nb = _tc_routing(x2d, gate_w)
    dest1 = dest[:, 0]
    dest2 = dest[:, 1]

    xs, ws = _sc_dispatch(x2d, dest1, dest2, wr1, wr2)
    outs = _tc_ffn(xs, ws, w1, w3, w2, be.reshape(NBLK), nb.reshape(1))
    y = _sc_combine_gather(outs, dest1, dest2)
    return y.reshape(orig_shape)
